# baseline (speedup 1.0000x reference)
import jax
import jax.numpy as jnp
from jax.experimental import pallas as pl
from jax.experimental.pallas import tpu as pltpu

_B = 64
_TMEL = 2048
_NCH = 80
_TSRC = 512
_NROW = _B * _TMEL
_R = 4096
_NG = _NROW // _R


def _loss_body(mt, mp, mq, mm, pt, pp, et, ep, mmf, dt, ldp, sv,
               o_total, o_mel, o_post, o_dur, o_pitch, o_energy, acc):
    i = pl.program_id(0)

    @pl.when(i == 0)
    def _init():
        mmv = mmf[...]
        svv = sv[...]
        log_dur_trg = jnp.log(dt[...].astype(jnp.float32) + 1.0)
        acc[0] = 0.0
        acc[1] = 0.0
        acc[2] = jnp.sum(jnp.square(pp[...] - pt[...]) * mmv)
        acc[3] = jnp.sum(jnp.square(ep[...] - et[...]) * mmv)
        acc[4] = jnp.sum(jnp.square(ldp[...] - log_dur_trg) * svv)
        acc[5] = jnp.sum(mmv)
        acc[6] = jnp.sum(svv)

    t = mt[...]
    m = mm[...]
    acc[0] = acc[0] + jnp.sum(jnp.abs(mp[...] - t) * m)
    acc[1] = acc[1] + jnp.sum(jnp.abs(mq[...] - t) * m)

    @pl.when(i == _NG - 1)
    def _fin():
        n_mel = acc[5]
        n_src = acc[6]
        mel_loss = acc[0] / (n_mel * _NCH)
        post_loss = acc[1] / (n_mel * _NCH)
        pitch_loss = acc[2] / n_mel
        energy_loss = acc[3] / n_mel
        dur_loss = acc[4] / n_src
        o_mel[0] = mel_loss
        o_post[0] = post_loss
        o_dur[0] = dur_loss
        o_pitch[0] = pitch_loss
        o_energy[0] = energy_loss
        o_total[0] = mel_loss + post_loss + dur_loss + pitch_loss + energy_loss


def kernel(mel_trg, dur_trg, pitch_trg, energy_trg, mel_pred,
           mel_postnet_pred, log_dur_pred, pitch_pred, energy_pred,
           src_mask, mel_mask):
    mt = mel_trg.reshape(_NROW, _NCH)
    mp = mel_pred.reshape(_NROW, _NCH)
    mq = mel_postnet_pred.reshape(_NROW, _NCH)
    mm = mel_mask.reshape(_NROW, 1).astype(jnp.float32)
    mmf = mel_mask.reshape(_NROW // 128, 128).astype(jnp.float32)
    sv = jnp.logical_not(src_mask).reshape(_B * _TSRC // 128, 128).astype(jnp.float32)
    pt = pitch_trg.reshape(_NROW // 128, 128)
    pp = pitch_pred.reshape(_NROW // 128, 128)
    et = energy_trg.reshape(_NROW // 128, 128)
    ep = energy_pred.reshape(_NROW // 128, 128)
    dt = dur_trg.reshape(_B * _TSRC // 128, 128)
    ldp = log_dur_pred.reshape(_B * _TSRC // 128, 128)

    full = lambda shape: pl.BlockSpec(shape, lambda i: (0, 0))
    out_spec = pl.BlockSpec(memory_space=pltpu.SMEM)
    outs = pl.pallas_call(
        _loss_body,
        grid=(_NG,),
        in_specs=[
            pl.BlockSpec((_R, _NCH), lambda i: (i, 0)),
            pl.BlockSpec((_R, _NCH), lambda i: (i, 0)),
            pl.BlockSpec((_R, _NCH), lambda i: (i, 0)),
            pl.BlockSpec((_R, 1), lambda i: (i, 0)),
            full((_NROW // 128, 128)),
            full((_NROW // 128, 128)),
            full((_NROW // 128, 128)),
            full((_NROW // 128, 128)),
            full((_NROW // 128, 128)),
            full((_B * _TSRC // 128, 128)),
            full((_B * _TSRC // 128, 128)),
            full((_B * _TSRC // 128, 128)),
        ],
        out_specs=[out_spec] * 6,
        out_shape=[jax.ShapeDtypeStruct((1,), jnp.float32)] * 6,
        scratch_shapes=[pltpu.SMEM((8,), jnp.float32)],
    )(mt, mp, mq, mm, pt, pp, et, ep, mmf, dt, ldp, sv)

    total, mel, post, dur, pitch, energy = [o[0] for o in outs]
    return (total, mel, post, dur, pitch, energy)
